# baseline (speedup 1.0000x reference)
import functools

import jax
import jax.numpy as jnp
from jax import lax
from jax.experimental import pallas as pl
from jax.experimental.pallas import tpu as pltpu
from jax.experimental.pallas import tpu_sc as plsc

N = 10000
E = 320000
MAX_NB = 32
H = 128
NMOLS_PAD = 512

NC, NS = 2, 16
NW = NC * NS


def _mm_body(relu_in, x_ref, w_ref, o_ref):
    x = x_ref[...]
    if relu_in:
        x = jnp.maximum(x, 0.0)
    o_ref[...] = jnp.dot(x, w_ref[...], preferred_element_type=jnp.float32)


def _tc_matmul(x, w, relu_in, block_rows=2000):
    m, k = x.shape
    _, n = w.shape
    grid = m // block_rows
    return pl.pallas_call(
        functools.partial(_mm_body, relu_in),
        grid=(grid,),
        in_specs=[
            pl.BlockSpec((block_rows, k), lambda i: (i, 0)),
            pl.BlockSpec((k, n), lambda i: (0, 0)),
        ],
        out_specs=pl.BlockSpec((block_rows, n), lambda i: (i, 0)),
        out_shape=jax.ShapeDtypeStruct((m, n), jnp.float32),
        compiler_params=pltpu.CompilerParams(
            dimension_semantics=("parallel",)),
    )(x, w)


N_PAD = 10240
GS_ATOMS = N_PAD // NW
GS_BLOCKS = GS_ATOMS // 4


def _make_gathersum(apply_relu):
    mesh = plsc.VectorSubcoreMesh(core_axis_name="c", subcore_axis_name="s")

    @functools.partial(
        pl.kernel,
        out_type=jax.ShapeDtypeStruct((N_PAD, H), jnp.float32),
        mesh=mesh,
        scratch_types=[
            pltpu.VMEM((GS_BLOCKS, 128), jnp.int32),
            pltpu.VMEM((128, H), jnp.float32),
            pltpu.VMEM((GS_ATOMS, H), jnp.float32),
            pltpu.SemaphoreType.DMA,
        ],
    )
    def gsum(mw_hbm, a2b_hbm, a_hbm, idx_v, rows_v, aout_v, sem):
        wid = lax.axis_index("s") * NC + lax.axis_index("c")
        base = wid * GS_ATOMS
        pltpu.sync_copy(a2b_hbm.at[pl.ds(wid * GS_BLOCKS, GS_BLOCKS)], idx_v)

        def block_body(b, _):
            pltpu.async_copy(mw_hbm.at[idx_v.at[b]], rows_v, sem).wait()
            for j in range(4):
                for c in range(H // 16):
                    sl = pl.ds(c * 16, 16)
                    r0 = rows_v[j * MAX_NB, sl]
                    if apply_relu:
                        r0 = jnp.maximum(r0, 0.0)
                    acc = r0
                    for r in range(1, MAX_NB):
                        v = rows_v[j * MAX_NB + r, sl]
                        if apply_relu:
                            v = jnp.maximum(v, 0.0)
                        acc = acc + v
                    aout_v[b * 4 + j, sl] = acc
            return 0

        lax.fori_loop(0, GS_BLOCKS, block_body, 0, unroll=False)
        pltpu.sync_copy(aout_v, a_hbm.at[pl.ds(base, GS_ATOMS)])

    return gsum


_gathersum = _make_gathersum(False)
_gathersum_relu = _make_gathersum(True)


CB_STRIDE = 10000
CB_EDGES = 10240
CB_BLOCKS = CB_EDGES // 128


def _make_combine():
    mesh = plsc.VectorSubcoreMesh(core_axis_name="c", subcore_axis_name="s")

    @functools.partial(
        pl.kernel,
        out_type=jax.ShapeDtypeStruct((E, H), jnp.float32),
        mesh=mesh,
        scratch_types=[
            pltpu.VMEM((CB_EDGES,), jnp.int32),
            pltpu.VMEM((CB_EDGES,), jnp.int32),
            pltpu.VMEM((128, H), jnp.float32),
            pltpu.VMEM((128, H), jnp.float32),
            pltpu.VMEM((128, H), jnp.float32),
            pltpu.VMEM((128, H), jnp.float32),
            pltpu.SemaphoreType.DMA,
            pltpu.SemaphoreType.DMA,
            pltpu.SemaphoreType.DMA,
        ],
    )
    def combine(inp_hbm, a_hbm, mw_hbm, b2a_hbm, b2revb_hbm, out_hbm,
                aidx_v, ridx_v, arows_v, mrows_v, irows_v, orows_v,
                sem_a, sem_m, sem_i):
        wid = lax.axis_index("s") * NC + lax.axis_index("c")
        base = jnp.minimum(wid * CB_STRIDE, E - CB_EDGES)
        pltpu.sync_copy(b2a_hbm.at[pl.ds(base, CB_EDGES)], aidx_v)
        pltpu.sync_copy(b2revb_hbm.at[pl.ds(base, CB_EDGES)], ridx_v)

        def block_body(b, _):
            e0 = base + b * 128
            cp_a = pltpu.async_copy(
                a_hbm.at[aidx_v.at[pl.ds(b * 128, 128)]], arows_v, sem_a)
            cp_m = pltpu.async_copy(
                mw_hbm.at[ridx_v.at[pl.ds(b * 128, 128)]], mrows_v, sem_m)
            cp_i = pltpu.async_copy(
                inp_hbm.at[pl.ds(e0, 128)], irows_v, sem_i)
            cp_a.wait()
            cp_m.wait()
            cp_i.wait()
            for r in range(128):
                for c in range(H // 16):
                    sl = pl.ds(c * 16, 16)
                    orows_v[r, sl] = irows_v[r, sl] + arows_v[r, sl] - mrows_v[r, sl]
            pltpu.sync_copy(orows_v, out_hbm.at[pl.ds(e0, 128)])
            return 0

        lax.fori_loop(0, CB_BLOCKS, block_body, 0, unroll=False)

    return combine


_combine = _make_combine()


RO_BLOCK = 2000
RO_GRID = N // RO_BLOCK


def _ro_body(fa_ref, a3_ref, wo1_ref, wo2_ref, bo_ref, mid_ref, out_ref,
             msum_ref, cnt_ref):
    i = pl.program_id(0)

    @pl.when(i == 0)
    def _():
        msum_ref[...] = jnp.zeros_like(msum_ref)
        cnt_ref[...] = jnp.zeros_like(cnt_ref)

    h = jnp.dot(fa_ref[...], wo1_ref[...], preferred_element_type=jnp.float32)
    h = h + jnp.dot(a3_ref[...], wo2_ref[...],
                    preferred_element_type=jnp.float32)
    h = jnp.maximum(h + bo_ref[...], 0.0)

    mid_row = mid_ref[0]
    mask_t = (lax.broadcasted_iota(jnp.int32, (NMOLS_PAD, RO_BLOCK), 0)
              == mid_row).astype(jnp.float32)
    msum_ref[...] += jnp.dot(mask_t, h, preferred_element_type=jnp.float32)
    cnt_ref[...] += jnp.broadcast_to(
        jnp.sum(mask_t, axis=1, keepdims=True), (NMOLS_PAD, H))

    @pl.when(i == RO_GRID - 1)
    def _():
        out_ref[...] = msum_ref[...] / jnp.maximum(cnt_ref[...], 1.0)


def _tc_readout(f_atoms, a3, wo1, wo2, bo, mol_ids_2d):
    return pl.pallas_call(
        _ro_body,
        grid=(RO_GRID,),
        in_specs=[
            pl.BlockSpec((RO_BLOCK, H), lambda i: (i, 0)),
            pl.BlockSpec((RO_BLOCK, H), lambda i: (i, 0)),
            pl.BlockSpec((H, H), lambda i: (0, 0)),
            pl.BlockSpec((H, H), lambda i: (0, 0)),
            pl.BlockSpec((1, H), lambda i: (0, 0)),
            pl.BlockSpec((1, 1, RO_BLOCK), lambda i: (i, 0, 0)),
        ],
        out_specs=pl.BlockSpec((NMOLS_PAD, H), lambda i: (0, 0)),
        out_shape=jax.ShapeDtypeStruct((NMOLS_PAD, H), jnp.float32),
        scratch_shapes=[
            pltpu.VMEM((NMOLS_PAD, H), jnp.float32),
            pltpu.VMEM((NMOLS_PAD, H), jnp.float32),
        ],
        compiler_params=pltpu.CompilerParams(
            dimension_semantics=("arbitrary",)),
    )(f_atoms, a3, wo1, wo2, bo, mol_ids_2d)


def kernel(f_atoms, f_bonds, a2b, b2a, b2revb, mol_ids, W_i, W_h, W_o, b_o):
    a2b = a2b.astype(jnp.int32)
    b2a = b2a.astype(jnp.int32)
    b2revb = b2revb.astype(jnp.int32)
    a2b_pad = jnp.zeros((N_PAD, MAX_NB), jnp.int32).at[:N].set(a2b)
    a2b_r = a2b_pad.reshape(N_PAD * MAX_NB // 128, 128)

    inp = _tc_matmul(f_bonds, W_i, relu_in=False)

    pre = inp
    for _ in range(2):
        mw = _tc_matmul(pre, W_h, relu_in=True)
        a_sum = _gathersum(mw, a2b_r)
        pre = _combine(inp, a_sum, mw, b2a, b2revb)

    a3 = _gathersum_relu(pre, a2b_r)[:N]

    mol_ids_3d = mol_ids.astype(jnp.int32).reshape(RO_GRID, 1, RO_BLOCK)
    wo1 = W_o[:H]
    wo2 = W_o[H:]
    bo = b_o.reshape(1, H)
    mol_vecs = _tc_readout(f_atoms, a3, wo1, wo2, bo, mol_ids_3d)
    return mol_vecs[:500]


# speedup vs baseline: 1.5004x; 1.5004x over previous
import functools

import jax
import jax.numpy as jnp
from jax import lax
from jax.experimental import pallas as pl
from jax.experimental.pallas import tpu as pltpu
from jax.experimental.pallas import tpu_sc as plsc

N = 10000
E = 320000
MAX_NB = 32
H = 128
NMOLS_PAD = 512

NC, NS = 2, 16
NW = NC * NS


def _mm_body(relu_in, x_ref, w_ref, o_ref):
    x = x_ref[...]
    if relu_in:
        x = jnp.maximum(x, 0.0)
    o_ref[...] = jnp.dot(x, w_ref[...], preferred_element_type=jnp.float32)


def _tc_matmul(x, w, relu_in, block_rows=2000):
    m, k = x.shape
    _, n = w.shape
    grid = m // block_rows
    return pl.pallas_call(
        functools.partial(_mm_body, relu_in),
        grid=(grid,),
        in_specs=[
            pl.BlockSpec((block_rows, k), lambda i: (i, 0)),
            pl.BlockSpec((k, n), lambda i: (0, 0)),
        ],
        out_specs=pl.BlockSpec((block_rows, n), lambda i: (i, 0)),
        out_shape=jax.ShapeDtypeStruct((m, n), jnp.float32),
        compiler_params=pltpu.CompilerParams(
            dimension_semantics=("parallel",)),
    )(x, w)


N_PAD = 10240
GS_ATOMS = N_PAD // NW
GS_BLOCKS = GS_ATOMS // 4


def _make_gathersum(apply_relu):
    mesh = plsc.VectorSubcoreMesh(core_axis_name="c", subcore_axis_name="s")

    @functools.partial(
        pl.kernel,
        out_type=jax.ShapeDtypeStruct((N_PAD, H), jnp.float32),
        mesh=mesh,
        scratch_types=[
            pltpu.VMEM((GS_BLOCKS, 128), jnp.int32),
            pltpu.VMEM((128, H), jnp.float32),
            pltpu.VMEM((128, H), jnp.float32),
            pltpu.VMEM((GS_ATOMS, H), jnp.float32),
            pltpu.SemaphoreType.DMA,
            pltpu.SemaphoreType.DMA,
        ],
    )
    def gsum(mw_hbm, a2b_hbm, a_hbm, idx_v, rows0_v, rows1_v, aout_v,
             sem0, sem1):
        wid = lax.axis_index("s") * NC + lax.axis_index("c")
        base = wid * GS_ATOMS
        pltpu.sync_copy(a2b_hbm.at[pl.ds(wid * GS_BLOCKS, GS_BLOCKS)], idx_v)

        def issue(b, rows_v, sem):
            pltpu.async_copy(mw_hbm.at[idx_v.at[b]], rows_v, sem)

        def drain(rows_v, sem):
            pltpu.make_async_copy(mw_hbm.at[pl.ds(0, 128)], rows_v, sem).wait()

        def process(b, rows_v):
            for j in range(4):
                for c in range(H // 16):
                    sl = pl.ds(c * 16, 16)
                    r0 = rows_v[j * MAX_NB, sl]
                    if apply_relu:
                        r0 = jnp.maximum(r0, 0.0)
                    acc = r0
                    for r in range(1, MAX_NB):
                        v = rows_v[j * MAX_NB + r, sl]
                        if apply_relu:
                            v = jnp.maximum(v, 0.0)
                        acc = acc + v
                    aout_v[b * 4 + j, sl] = acc

        issue(0, rows0_v, sem0)

        def pair_body(i, _):
            b0 = 2 * i
            issue(b0 + 1, rows1_v, sem1)
            drain(rows0_v, sem0)
            process(b0, rows0_v)

            @pl.when(i < GS_BLOCKS // 2 - 1)
            def _():
                issue(b0 + 2, rows0_v, sem0)

            drain(rows1_v, sem1)
            process(b0 + 1, rows1_v)
            return 0

        lax.fori_loop(0, GS_BLOCKS // 2, pair_body, 0, unroll=False)
        pltpu.sync_copy(aout_v, a_hbm.at[pl.ds(base, GS_ATOMS)])

    return gsum


_gathersum = _make_gathersum(False)
_gathersum_relu = _make_gathersum(True)


CB_STRIDE = 10000
CB_EDGES = 10240
CB_EB = 64
CB_BLOCKS = CB_EDGES // CB_EB


def _make_combine():
    mesh = plsc.VectorSubcoreMesh(core_axis_name="c", subcore_axis_name="s")

    @functools.partial(
        pl.kernel,
        out_type=jax.ShapeDtypeStruct((E, H), jnp.float32),
        mesh=mesh,
        scratch_types=[
            pltpu.VMEM((CB_EDGES,), jnp.int32),
            pltpu.VMEM((CB_EDGES,), jnp.int32),
            pltpu.VMEM((CB_EB, H), jnp.float32),
            pltpu.VMEM((CB_EB, H), jnp.float32),
            pltpu.VMEM((CB_EB, H), jnp.float32),
            pltpu.VMEM((CB_EB, H), jnp.float32),
            pltpu.VMEM((CB_EB, H), jnp.float32),
            pltpu.VMEM((CB_EB, H), jnp.float32),
            pltpu.VMEM((CB_EB, H), jnp.float32),
            pltpu.VMEM((CB_EB, H), jnp.float32),
            pltpu.SemaphoreType.DMA, pltpu.SemaphoreType.DMA,
            pltpu.SemaphoreType.DMA, pltpu.SemaphoreType.DMA,
            pltpu.SemaphoreType.DMA, pltpu.SemaphoreType.DMA,
            pltpu.SemaphoreType.DMA, pltpu.SemaphoreType.DMA,
        ],
    )
    def combine(inp_hbm, a_hbm, mw_hbm, b2a_hbm, b2revb_hbm, out_hbm,
                aidx_v, ridx_v,
                ar0, mr0, ir0, or0, ar1, mr1, ir1, or1,
                sa0, sm0, si0, so0, sa1, sm1, si1, so1):
        wid = lax.axis_index("s") * NC + lax.axis_index("c")
        base = jnp.minimum(wid * CB_STRIDE, E - CB_EDGES)
        pltpu.sync_copy(b2a_hbm.at[pl.ds(base, CB_EDGES)], aidx_v)
        pltpu.sync_copy(b2revb_hbm.at[pl.ds(base, CB_EDGES)], ridx_v)

        def issue_in(b, ar, mr, ir, sa, sm, si):
            off = b * CB_EB
            pltpu.async_copy(a_hbm.at[aidx_v.at[pl.ds(off, CB_EB)]], ar, sa)
            pltpu.async_copy(mw_hbm.at[ridx_v.at[pl.ds(off, CB_EB)]], mr, sm)
            pltpu.async_copy(inp_hbm.at[pl.ds(base + off, CB_EB)], ir, si)

        def drain_in(ar, mr, ir, sa, sm, si):
            pltpu.make_async_copy(a_hbm.at[pl.ds(0, CB_EB)], ar, sa).wait()
            pltpu.make_async_copy(mw_hbm.at[pl.ds(0, CB_EB)], mr, sm).wait()
            pltpu.make_async_copy(inp_hbm.at[pl.ds(0, CB_EB)], ir, si).wait()

        def drain_out(orv, so):
            pltpu.make_async_copy(orv, out_hbm.at[pl.ds(0, CB_EB)], so).wait()

        def compute(ar, mr, ir, orv):
            def row_body(r, _):
                for c in range(H // 16):
                    sl = pl.ds(c * 16, 16)
                    orv[r, sl] = ir[r, sl] + ar[r, sl] - mr[r, sl]
                return 0
            lax.fori_loop(0, CB_EB, row_body, 0, unroll=4)

        issue_in(0, ar0, mr0, ir0, sa0, sm0, si0)

        def pair_body(i, _):
            b0 = 2 * i
            issue_in(b0 + 1, ar1, mr1, ir1, sa1, sm1, si1)
            drain_in(ar0, mr0, ir0, sa0, sm0, si0)

            @pl.when(i > 0)
            def _():
                drain_out(or0, so0)

            compute(ar0, mr0, ir0, or0)
            pltpu.async_copy(or0, out_hbm.at[pl.ds(base + b0 * CB_EB, CB_EB)],
                             so0)

            @pl.when(i < CB_BLOCKS // 2 - 1)
            def _():
                issue_in(b0 + 2, ar0, mr0, ir0, sa0, sm0, si0)

            drain_in(ar1, mr1, ir1, sa1, sm1, si1)

            @pl.when(i > 0)
            def _():
                drain_out(or1, so1)

            compute(ar1, mr1, ir1, or1)
            pltpu.async_copy(
                or1, out_hbm.at[pl.ds(base + (b0 + 1) * CB_EB, CB_EB)], so1)
            return 0

        lax.fori_loop(0, CB_BLOCKS // 2, pair_body, 0, unroll=False)
        drain_out(or0, so0)
        drain_out(or1, so1)

    return combine


_combine = _make_combine()


RO_BLOCK = 2000
RO_GRID = N // RO_BLOCK


def _ro_body(fa_ref, a3_ref, wo1_ref, wo2_ref, bo_ref, mid_ref, out_ref,
             msum_ref, cnt_ref):
    i = pl.program_id(0)

    @pl.when(i == 0)
    def _():
        msum_ref[...] = jnp.zeros_like(msum_ref)
        cnt_ref[...] = jnp.zeros_like(cnt_ref)

    h = jnp.dot(fa_ref[...], wo1_ref[...], preferred_element_type=jnp.float32)
    h = h + jnp.dot(a3_ref[...], wo2_ref[...],
                    preferred_element_type=jnp.float32)
    h = jnp.maximum(h + bo_ref[...], 0.0)

    mid_row = mid_ref[0]
    mask_t = (lax.broadcasted_iota(jnp.int32, (NMOLS_PAD, RO_BLOCK), 0)
              == mid_row).astype(jnp.float32)
    msum_ref[...] += jnp.dot(mask_t, h, preferred_element_type=jnp.float32)
    cnt_ref[...] += jnp.broadcast_to(
        jnp.sum(mask_t, axis=1, keepdims=True), (NMOLS_PAD, H))

    @pl.when(i == RO_GRID - 1)
    def _():
        out_ref[...] = msum_ref[...] / jnp.maximum(cnt_ref[...], 1.0)


def _tc_readout(f_atoms, a3, wo1, wo2, bo, mol_ids_2d):
    return pl.pallas_call(
        _ro_body,
        grid=(RO_GRID,),
        in_specs=[
            pl.BlockSpec((RO_BLOCK, H), lambda i: (i, 0)),
            pl.BlockSpec((RO_BLOCK, H), lambda i: (i, 0)),
            pl.BlockSpec((H, H), lambda i: (0, 0)),
            pl.BlockSpec((H, H), lambda i: (0, 0)),
            pl.BlockSpec((1, H), lambda i: (0, 0)),
            pl.BlockSpec((1, 1, RO_BLOCK), lambda i: (i, 0, 0)),
        ],
        out_specs=pl.BlockSpec((NMOLS_PAD, H), lambda i: (0, 0)),
        out_shape=jax.ShapeDtypeStruct((NMOLS_PAD, H), jnp.float32),
        scratch_shapes=[
            pltpu.VMEM((NMOLS_PAD, H), jnp.float32),
            pltpu.VMEM((NMOLS_PAD, H), jnp.float32),
        ],
        compiler_params=pltpu.CompilerParams(
            dimension_semantics=("arbitrary",)),
    )(f_atoms, a3, wo1, wo2, bo, mol_ids_2d)


def kernel(f_atoms, f_bonds, a2b, b2a, b2revb, mol_ids, W_i, W_h, W_o, b_o):
    a2b = a2b.astype(jnp.int32)
    b2a = b2a.astype(jnp.int32)
    b2revb = b2revb.astype(jnp.int32)
    a2b_pad = jnp.zeros((N_PAD, MAX_NB), jnp.int32).at[:N].set(a2b)
    a2b_r = a2b_pad.reshape(N_PAD * MAX_NB // 128, 128)

    inp = _tc_matmul(f_bonds, W_i, relu_in=False)

    pre = inp
    for _ in range(2):
        mw = _tc_matmul(pre, W_h, relu_in=True)
        a_sum = _gathersum(mw, a2b_r)
        pre = _combine(inp, a_sum, mw, b2a, b2revb)

    a3 = _gathersum_relu(pre, a2b_r)[:N]

    mol_ids_3d = mol_ids.astype(jnp.int32).reshape(RO_GRID, 1, RO_BLOCK)
    wo1 = W_o[:H]
    wo2 = W_o[H:]
    bo = b_o.reshape(1, H)
    mol_vecs = _tc_readout(f_atoms, a3, wo1, wo2, bo, mol_ids_3d)
    return mol_vecs[:500]
